# Initial kernel scaffold; baseline (speedup 1.0000x reference)
import jax
import jax.numpy as jnp
from jax.experimental import pallas as pl


def kernel(x, edge_index, edge_weight, batch, W1, b1, W2, b2, lw1, lb1, lw2, lb2):
    raise NotImplementedError("write your pallas kernel here")


# speedup vs baseline: 2.9325x; 2.9325x over previous
import functools

import jax
import jax.numpy as jnp
from jax import lax
from jax.experimental import pallas as pl
from jax.experimental.pallas import tpu as pltpu

N = 10000
E = 320000
IN = 128
H = 256
OUT = 64
G = 128


def _prep_body(deg_ref, x_ref, dinv_ref, xs_ref):
    deg = deg_ref[...]
    dinv = jnp.where(deg > 0, lax.rsqrt(deg), 0.0)
    dinv_ref[...] = dinv
    xs_ref[...] = x_ref[...] * dinv[:, None]


def _prep(deg, x):
    return pl.pallas_call(
        _prep_body,
        out_shape=(
            jax.ShapeDtypeStruct((N,), jnp.float32),
            jax.ShapeDtypeStruct((N, IN), jnp.float32),
        ),
    )(deg, x)


def _layer1_body(y1_ref, x_ref, dinv_ref, W1_ref, b1_ref, h1_ref, h1s_ref):
    dinv = dinv_ref[...]
    agg = y1_ref[...] * dinv[:, None] + x_ref[...] * (dinv * dinv)[:, None]
    h = jnp.dot(agg, W1_ref[...], preferred_element_type=jnp.float32)
    h = jnp.maximum(h + b1_ref[...][None, :], 0.0)
    h1_ref[...] = h
    h1s_ref[...] = h * dinv[:, None]


def _layer1(y1, x, dinv, W1, b1):
    return pl.pallas_call(
        _layer1_body,
        out_shape=(
            jax.ShapeDtypeStruct((N, H), jnp.float32),
            jax.ShapeDtypeStruct((N, H), jnp.float32),
        ),
    )(y1, x, dinv, W1, b1)


def _layer2_body(y2_ref, h1_ref, dinv_ref, batch_ref, W2_ref, b2_ref,
                 lw1_ref, lb1_ref, lw2_ref, lb2_ref, xo_ref, xfea_ref):
    dinv = dinv_ref[...]
    agg = y2_ref[...] * dinv[:, None] + h1_ref[...] * (dinv * dinv)[:, None]
    h = jnp.dot(agg, W2_ref[...], preferred_element_type=jnp.float32)
    h = jnp.maximum(h + b2_ref[...][None, :], 0.0)
    gids = lax.broadcasted_iota(jnp.int32, (N, G), 1)
    onehot = (batch_ref[...][:, None] == gids).astype(jnp.float32)
    sums = jnp.dot(onehot.T, h, preferred_element_type=jnp.float32)
    cnt = jnp.sum(onehot, axis=0)
    pooled = sums / jnp.maximum(cnt, 1.0)[:, None]
    xfea = jnp.dot(pooled, lw1_ref[...], preferred_element_type=jnp.float32)
    xfea = xfea + lb1_ref[...][None, :]
    xo = jnp.dot(jnp.maximum(xfea, 0.0), lw2_ref[...],
                 preferred_element_type=jnp.float32) + lb2_ref[...][None, :]
    xo_ref[...] = xo
    xfea_ref[...] = xfea


def _layer2(y2, h1, dinv, batch, W2, b2, lw1, lb1, lw2, lb2):
    return pl.pallas_call(
        _layer2_body,
        out_shape=(
            jax.ShapeDtypeStruct((G, OUT), jnp.float32),
            jax.ShapeDtypeStruct((G, IN), jnp.float32),
        ),
    )(y2, h1, dinv, batch, W2, b2, lw1, lb1, lw2, lb2)


def kernel(x, edge_index, edge_weight, batch, W1, b1, W2, b2, lw1, lb1, lw2, lb2):
    s = edge_index[0]
    d = edge_index[1]
    ew = edge_weight

    deg = jnp.ones((N,), jnp.float32).at[d].add(ew)
    dinv, xs = _prep(deg, x)

    y1 = jnp.zeros((N, IN), jnp.float32).at[d].add(xs[s] * ew[:, None])
    h1, h1s = _layer1(y1, x, dinv, W1, b1)

    y2 = jnp.zeros((N, H), jnp.float32).at[d].add(h1s[s] * ew[:, None])
    xo, xfea = _layer2(y2, h1, dinv, batch.astype(jnp.int32), W2, b2,
                       lw1, lb1, lw2, lb2)
    return (xo, xfea)


# speedup vs baseline: 16.2481x; 5.5407x over previous
import functools

import jax
import jax.numpy as jnp
from jax import lax
from jax.experimental import pallas as pl
from jax.experimental.pallas import tpu as pltpu
from jax.experimental.pallas import tpu_sc as plsc

N = 10000
NP = 10240
E = 320000
EP = 323584
IN = 128
H = 256
OUT = 64
G = 128

NC = 2
NS = 16
RPT = NP // NS

_MESH = plsc.VectorSubcoreMesh(core_axis_name="c", subcore_axis_name="s")


def _sc_deg_body(d3, ew3, out, d_v, ew_v, z_v, acc_sp):
    cid = lax.axis_index("c")
    sid = lax.axis_index("s")
    w = sid * NC + cid
    def zb(i, c):
        z_v[pl.ds(i * 16, 16)] = jnp.zeros((16,), jnp.float32)
        return c
    lax.fori_loop(0, RPT // 16, zb, 0)
    pltpu.sync_copy(z_v, acc_sp.at[pl.ds(sid * RPT, RPT)])
    pltpu.sync_copy(d3.at[w], d_v)
    pltpu.sync_copy(ew3.at[w], ew_v)
    plsc.subcore_barrier()

    def chunk(j, c):
        pltpu.sync_copy(ew_v.at[j], acc_sp.at[d_v.at[j]], add=True)
        return c
    lax.fori_loop(0, EP // (32 * 128), chunk, 0)
    plsc.subcore_barrier()
    pltpu.sync_copy(acc_sp.at[pl.ds(sid * RPT, RPT)],
                    out.at[cid, pl.ds(sid * RPT, RPT)])


_sc_deg = functools.partial(
    pl.kernel,
    out_type=pltpu.HBM((NC, NP), jnp.float32),
    mesh=_MESH,
    scratch_types=[
        pltpu.VMEM((EP // (32 * 128), 128), jnp.int32),
        pltpu.VMEM((EP // (32 * 128), 128), jnp.float32),
        pltpu.VMEM((RPT,), jnp.float32),
        pltpu.VMEM_SHARED((NP,), jnp.float32),
    ],
)(_sc_deg_body)


def _make_sc_agg(split_edges):
    CH = EP // (32 * 128) if split_edges else EP // (16 * 128)
    ng = 8
    BLK = 32
    blocks = [(b, min(BLK, CH - b)) for b in range(0, CH, BLK)]

    def body(xs_flat, s3, d3, ew3, out, s_v, d_v, ew_v, rows_v, sem, acc_sp):
        cid = lax.axis_index("c")
        sid = lax.axis_index("s")

        def zrow(i, c):
            for g in range(ng):
                rows_v[i, pl.ds(g * 16, 16)] = jnp.zeros((16,), jnp.float32)
            return c
        lax.fori_loop(0, 128, zrow, 0)
        for k in range(RPT // 128):
            pltpu.sync_copy(rows_v,
                            acc_sp.at[pl.ds(sid * RPT + k * 128, 128)])
        plsc.subcore_barrier()

        for bst, bn in blocks:
            bsl = pl.ds(bst, bn)
            dsl = pl.ds(0, bn)
            if split_edges:
                w = sid * NC + cid
                pltpu.sync_copy(s3.at[w, bsl], s_v.at[dsl])
                pltpu.sync_copy(d3.at[w, bsl], d_v.at[dsl])
                pltpu.sync_copy(ew3.at[w, bsl], ew_v.at[dsl])
            else:
                pltpu.sync_copy(s3.at[cid, sid, bsl], s_v.at[dsl])
                pltpu.sync_copy(d3.at[sid, bsl], d_v.at[dsl])
                pltpu.sync_copy(ew3.at[sid, bsl], ew_v.at[dsl])

            def chunk(j, c):
                pltpu.async_copy(xs_flat.at[s_v.at[j]], rows_v, sem).wait()

                def scale(i16, c2):
                    ev = ew_v[j, pl.ds(i16 * 16, 16)]
                    for l in range(16):
                        e = ev[l]
                        i = i16 * 16 + l
                        for g in range(ng):
                            sl = pl.ds(g * 16, 16)
                            rows_v[i, sl] = rows_v[i, sl] * e
                    return c2
                lax.fori_loop(0, 8, scale, 0)
                pltpu.sync_copy(rows_v, acc_sp.at[d_v.at[j]], add=True)
                return c
            lax.fori_loop(0, bn, chunk, 0)
        plsc.subcore_barrier()
        for k in range(RPT // 128):
            sl = pl.ds(sid * RPT + k * 128, 128)
            pltpu.sync_copy(acc_sp.at[sl], out.at[cid, sl])

    return pl.kernel(
        body,
        out_type=pltpu.HBM((NC, NP, 128), jnp.float32),
        mesh=_MESH,
        scratch_types=[
            pltpu.VMEM((BLK, 128), jnp.int32),
            pltpu.VMEM((BLK, 128), jnp.int32),
            pltpu.VMEM((BLK, 128), jnp.float32),
            pltpu.VMEM((128, 128), jnp.float32),
            pltpu.SemaphoreType.DMA,
            pltpu.VMEM_SHARED((NP, 128), jnp.float32),
        ],
    )


_sc_agg1 = _make_sc_agg(True)
_sc_agg2 = _make_sc_agg(False)


def _prep_body(degp_ref, x_ref, dinv_ref, xs_ref):
    deg = degp_ref[0, :N] + degp_ref[1, :N] + 1.0
    dinv = lax.rsqrt(deg)
    dinv_ref[...] = dinv
    xs_ref[...] = x_ref[...] * dinv[:, None]


def _prep(degp, x):
    return pl.pallas_call(
        _prep_body,
        out_shape=(
            jax.ShapeDtypeStruct((N,), jnp.float32),
            jax.ShapeDtypeStruct((N, IN), jnp.float32),
        ),
    )(degp, x)


def _layer1_body(y1_ref, x_ref, dinv_ref, W1_ref, b1_ref, h1_ref, h1s_ref):
    dinv = dinv_ref[...]
    y1 = y1_ref[0, :N] + y1_ref[1, :N]
    agg = y1 * dinv[:, None] + x_ref[...] * (dinv * dinv)[:, None]
    h = jnp.dot(agg, W1_ref[...], preferred_element_type=jnp.float32)
    h = jnp.maximum(h + b1_ref[...][None, :], 0.0)
    h1_ref[...] = h
    hs = h * dinv[:, None]
    h1s_ref[0] = hs[:, : H // 2]
    h1s_ref[1] = hs[:, H // 2:]


def _layer1(y1t, x, dinv, W1, b1):
    return pl.pallas_call(
        _layer1_body,
        out_shape=(
            jax.ShapeDtypeStruct((N, H), jnp.float32),
            jax.ShapeDtypeStruct((NC, N, H // 2), jnp.float32),
        ),
    )(y1t, x, dinv, W1, b1)


def _layer2_body(y2_ref, h1_ref, dinv_ref, batch_ref, W2_ref, b2_ref,
                 lw1_ref, lb1_ref, lw2_ref, lb2_ref, xo_ref, xfea_ref):
    dinv = dinv_ref[...]
    y2 = jnp.concatenate([y2_ref[0, :N], y2_ref[1, :N]], axis=1)
    agg = y2 * dinv[:, None] + h1_ref[...] * (dinv * dinv)[:, None]
    h = jnp.dot(agg, W2_ref[...], preferred_element_type=jnp.float32)
    h = jnp.maximum(h + b2_ref[...][None, :], 0.0)
    gids = lax.broadcasted_iota(jnp.int32, (N, G), 1)
    onehot = (batch_ref[...][:, None] == gids).astype(jnp.float32)
    sums = jnp.dot(onehot.T, h, preferred_element_type=jnp.float32)
    cnt = jnp.sum(onehot, axis=0)
    pooled = sums / jnp.maximum(cnt, 1.0)[:, None]
    xfea = jnp.dot(pooled, lw1_ref[...], preferred_element_type=jnp.float32)
    xfea = xfea + lb1_ref[...][None, :]
    xo = jnp.dot(jnp.maximum(xfea, 0.0), lw2_ref[...],
                 preferred_element_type=jnp.float32) + lb2_ref[...][None, :]
    xo_ref[...] = xo
    xfea_ref[...] = xfea


def _layer2(y2t, h1, dinv, batch, W2, b2, lw1, lb1, lw2, lb2):
    return pl.pallas_call(
        _layer2_body,
        out_shape=(
            jax.ShapeDtypeStruct((G, OUT), jnp.float32),
            jax.ShapeDtypeStruct((G, IN), jnp.float32),
        ),
    )(y2t, h1, dinv, batch, W2, b2, lw1, lb1, lw2, lb2)


def kernel(x, edge_index, edge_weight, batch, W1, b1, W2, b2, lw1, lb1, lw2, lb2):
    s = edge_index[0].astype(jnp.int32)
    d = edge_index[1].astype(jnp.int32)
    ew = edge_weight

    pad = EP - E
    padidx = jnp.arange(pad, dtype=jnp.int32) % N
    s_p = jnp.concatenate([s, padidx])
    d_p = jnp.concatenate([d, padidx])
    ew_p = jnp.concatenate([ew, jnp.zeros((pad,), jnp.float32)])

    CHD = EP // (32 * 128)
    s_deg = s_p.reshape(32, CHD, 128)
    d_deg = d_p.reshape(32, CHD, 128)
    ew_deg = ew_p.reshape(32, CHD, 128)
    CH = EP // (16 * 128)
    s3 = jnp.stack([s_p, s_p + N]).reshape(NC, NS, CH, 128)
    d3 = d_p.reshape(NS, CH, 128)
    ew3 = ew_p.reshape(NS, CH, 128)

    degp = _sc_deg(d_deg, ew_deg)
    dinv, xs = _prep(degp, x)

    y1t = _sc_agg1(xs, s_deg, d_deg, ew_deg)
    h1, h1s_t = _layer1(y1t, x, dinv, W1, b1)

    y2t = _sc_agg2(h1s_t.reshape(NC * N, H // 2), s3, d3, ew3)
    xo, xfea = _layer2(y2t, h1, dinv, batch.astype(jnp.int32), W2, b2,
                       lw1, lb1, lw2, lb2)
    return (xo, xfea)


# speedup vs baseline: 25.2475x; 1.5539x over previous
import functools

import jax
import jax.numpy as jnp
from jax import lax
from jax.experimental import pallas as pl
from jax.experimental.pallas import tpu as pltpu
from jax.experimental.pallas import tpu_sc as plsc

N = 10000
NP = 10240
E = 320000
EP = 323584
IN = 128
H = 256
OUT = 64
G = 128

NC = 2
NS = 16
RPT = NP // NS

_MESH = plsc.VectorSubcoreMesh(core_axis_name="c", subcore_axis_name="s")


def _sc_deg_body(d3, ew3, out, d_v, ew_v, z_v, acc_sp):
    cid = lax.axis_index("c")
    sid = lax.axis_index("s")
    w = sid * NC + cid
    def zb(i, c):
        z_v[pl.ds(i * 16, 16)] = jnp.zeros((16,), jnp.float32)
        return c
    lax.fori_loop(0, RPT // 16, zb, 0)
    pltpu.sync_copy(z_v, acc_sp.at[pl.ds(sid * RPT, RPT)])
    pltpu.sync_copy(d3.at[w], d_v)
    pltpu.sync_copy(ew3.at[w], ew_v)
    plsc.subcore_barrier()

    def chunk(j, c):
        pltpu.sync_copy(ew_v.at[j], acc_sp.at[d_v.at[j]], add=True)
        return c
    lax.fori_loop(0, EP // (32 * 128), chunk, 0)
    plsc.subcore_barrier()
    pltpu.sync_copy(acc_sp.at[pl.ds(sid * RPT, RPT)],
                    out.at[cid, pl.ds(sid * RPT, RPT)])


_sc_deg = functools.partial(
    pl.kernel,
    out_type=pltpu.HBM((NC, NP), jnp.float32),
    mesh=_MESH,
    scratch_types=[
        pltpu.VMEM((EP // (32 * 128), 128), jnp.int32),
        pltpu.VMEM((EP // (32 * 128), 128), jnp.float32),
        pltpu.VMEM((RPT,), jnp.float32),
        pltpu.VMEM_SHARED((NP,), jnp.float32),
    ],
)(_sc_deg_body)


def _make_sc_agg(split_edges):
    CH = EP // (32 * 128) if split_edges else EP // (16 * 128)
    ng = 8
    BLK = 32
    blocks = [(b, min(BLK, CH - b)) for b in range(0, CH, BLK)]

    def body(xs_flat, s3, d3, ew3, out, s_v, d_v, ew_v, rows_a, rows_b, sem,
             acc_sp):
        cid = lax.axis_index("c")
        sid = lax.axis_index("s")

        def zrow(i, c):
            for g in range(ng):
                rows_a[i, pl.ds(g * 16, 16)] = jnp.zeros((16,), jnp.float32)
            return c
        lax.fori_loop(0, 128, zrow, 0)
        for k in range(RPT // 128):
            pltpu.sync_copy(rows_a,
                            acc_sp.at[pl.ds(sid * RPT + k * 128, 128)])
        plsc.subcore_barrier()

        def scale_scatter(j, rows_v):
            def scale(i16, c2):
                ev = ew_v[j, pl.ds(i16 * 16, 16)]
                for l in range(16):
                    e = ev[l]
                    i = i16 * 16 + l
                    for g in range(ng):
                        sl = pl.ds(g * 16, 16)
                        rows_v[i, sl] = rows_v[i, sl] * e
                return c2
            lax.fori_loop(0, 8, scale, 0)
            pltpu.sync_copy(rows_v, acc_sp.at[d_v.at[j]], add=True)

        for bst, bn in blocks:
            bsl = pl.ds(bst, bn)
            dsl = pl.ds(0, bn)
            if split_edges:
                w = sid * NC + cid
                pltpu.sync_copy(s3.at[w, bsl], s_v.at[dsl])
                pltpu.sync_copy(d3.at[w, bsl], d_v.at[dsl])
                pltpu.sync_copy(ew3.at[w, bsl], ew_v.at[dsl])
            else:
                pltpu.sync_copy(s3.at[cid, sid, bsl], s_v.at[dsl])
                pltpu.sync_copy(d3.at[sid, bsl], d_v.at[dsl])
                pltpu.sync_copy(ew3.at[sid, bsl], ew_v.at[dsl])

            pltpu.async_copy(xs_flat.at[s_v.at[0]], rows_a, sem)

            def pair(t, c):
                j0 = t * 2

                @pl.when(j0 + 1 < bn)
                def _():
                    pltpu.async_copy(xs_flat.at[s_v.at[j0 + 1]], rows_b, sem)
                pltpu.make_async_copy(xs_flat.at[s_v.at[j0]], rows_a,
                                      sem).wait()
                scale_scatter(j0, rows_a)

                @pl.when(j0 + 2 < bn)
                def _():
                    pltpu.async_copy(xs_flat.at[s_v.at[j0 + 2]], rows_a, sem)

                @pl.when(j0 + 1 < bn)
                def _():
                    pltpu.make_async_copy(xs_flat.at[s_v.at[j0 + 1]], rows_b,
                                          sem).wait()
                    scale_scatter(j0 + 1, rows_b)
                return c
            lax.fori_loop(0, (bn + 1) // 2, pair, 0)
        plsc.subcore_barrier()
        for k in range(RPT // 128):
            sl = pl.ds(sid * RPT + k * 128, 128)
            pltpu.sync_copy(acc_sp.at[sl], out.at[cid, sl])

    return pl.kernel(
        body,
        out_type=pltpu.HBM((NC, NP, 128), jnp.float32),
        mesh=_MESH,
        scratch_types=[
            pltpu.VMEM((BLK, 128), jnp.int32),
            pltpu.VMEM((BLK, 128), jnp.int32),
            pltpu.VMEM((BLK, 128), jnp.float32),
            pltpu.VMEM((128, 128), jnp.float32),
            pltpu.VMEM((128, 128), jnp.float32),
            pltpu.SemaphoreType.DMA,
            pltpu.VMEM_SHARED((NP, 128), jnp.float32),
        ],
    )


_sc_agg1 = _make_sc_agg(True)
_sc_agg2 = _make_sc_agg(False)


def _prep_body(degp_ref, x_ref, dinv_ref, xs_ref):
    deg = degp_ref[0, :N] + degp_ref[1, :N] + 1.0
    dinv = lax.rsqrt(deg)
    dinv_ref[...] = dinv
    xs_ref[...] = x_ref[...] * dinv[:, None]


def _prep(degp, x):
    return pl.pallas_call(
        _prep_body,
        out_shape=(
            jax.ShapeDtypeStruct((N,), jnp.float32),
            jax.ShapeDtypeStruct((N, IN), jnp.float32),
        ),
    )(degp, x)


def _layer1_body(y1_ref, x_ref, dinv_ref, W1_ref, b1_ref, h1_ref, h1s_ref):
    dinv = dinv_ref[...]
    y1 = y1_ref[0, :N] + y1_ref[1, :N]
    agg = y1 * dinv[:, None] + x_ref[...] * (dinv * dinv)[:, None]
    h = jnp.dot(agg, W1_ref[...], preferred_element_type=jnp.float32)
    h = jnp.maximum(h + b1_ref[...][None, :], 0.0)
    h1_ref[...] = h
    hs = h * dinv[:, None]
    h1s_ref[0] = hs[:, : H // 2]
    h1s_ref[1] = hs[:, H // 2:]


def _layer1(y1t, x, dinv, W1, b1):
    return pl.pallas_call(
        _layer1_body,
        out_shape=(
            jax.ShapeDtypeStruct((N, H), jnp.float32),
            jax.ShapeDtypeStruct((NC, N, H // 2), jnp.float32),
        ),
    )(y1t, x, dinv, W1, b1)


def _layer2_body(y2_ref, h1_ref, dinv_ref, batch_ref, W2_ref, b2_ref,
                 lw1_ref, lb1_ref, lw2_ref, lb2_ref, xo_ref, xfea_ref):
    dinv = dinv_ref[...]
    y2 = jnp.concatenate([y2_ref[0, :N], y2_ref[1, :N]], axis=1)
    agg = y2 * dinv[:, None] + h1_ref[...] * (dinv * dinv)[:, None]
    h = jnp.dot(agg, W2_ref[...], preferred_element_type=jnp.float32)
    h = jnp.maximum(h + b2_ref[...][None, :], 0.0)
    gids = lax.broadcasted_iota(jnp.int32, (N, G), 1)
    onehot = (batch_ref[...][:, None] == gids).astype(jnp.float32)
    sums = jnp.dot(onehot.T, h, preferred_element_type=jnp.float32)
    cnt = jnp.sum(onehot, axis=0)
    pooled = sums / jnp.maximum(cnt, 1.0)[:, None]
    xfea = jnp.dot(pooled, lw1_ref[...], preferred_element_type=jnp.float32)
    xfea = xfea + lb1_ref[...][None, :]
    xo = jnp.dot(jnp.maximum(xfea, 0.0), lw2_ref[...],
                 preferred_element_type=jnp.float32) + lb2_ref[...][None, :]
    xo_ref[...] = xo
    xfea_ref[...] = xfea


def _layer2(y2t, h1, dinv, batch, W2, b2, lw1, lb1, lw2, lb2):
    return pl.pallas_call(
        _layer2_body,
        out_shape=(
            jax.ShapeDtypeStruct((G, OUT), jnp.float32),
            jax.ShapeDtypeStruct((G, IN), jnp.float32),
        ),
    )(y2t, h1, dinv, batch, W2, b2, lw1, lb1, lw2, lb2)


def kernel(x, edge_index, edge_weight, batch, W1, b1, W2, b2, lw1, lb1, lw2, lb2):
    s = edge_index[0].astype(jnp.int32)
    d = edge_index[1].astype(jnp.int32)
    ew = edge_weight

    pad = EP - E
    padidx = jnp.arange(pad, dtype=jnp.int32) % N
    s_p = jnp.concatenate([s, padidx])
    d_p = jnp.concatenate([d, padidx])
    ew_p = jnp.concatenate([ew, jnp.zeros((pad,), jnp.float32)])

    CHD = EP // (32 * 128)
    s_deg = s_p.reshape(32, CHD, 128)
    d_deg = d_p.reshape(32, CHD, 128)
    ew_deg = ew_p.reshape(32, CHD, 128)
    CH = EP // (16 * 128)
    s3 = jnp.stack([s_p, s_p + N]).reshape(NC, NS, CH, 128)
    d3 = d_p.reshape(NS, CH, 128)
    ew3 = ew_p.reshape(NS, CH, 128)

    degp = _sc_deg(d_deg, ew_deg)
    dinv, xs = _prep(degp, x)

    y1t = _sc_agg1(xs, s_deg, d_deg, ew_deg)
    h1, h1s_t = _layer1(y1t, x, dinv, W1, b1)

    y2t = _sc_agg2(h1s_t.reshape(NC * N, H // 2), s3, d3, ew3)
    xo, xfea = _layer2(y2t, h1, dinv, batch.astype(jnp.int32), W2, b2,
                       lw1, lb1, lw2, lb2)
    return (xo, xfea)
